# Initial kernel scaffold; baseline (speedup 1.0000x reference)
import jax
import jax.numpy as jnp
from jax.experimental import pallas as pl


def kernel(x_in, edge_index, deg, W_gc1, b_gc1, w_g1, b_g1, W_gc2, b_gc2, w_g2, b_g2, bn_gamma, bn_beta, bn_mean, bn_var, W_out, b_out):
    raise NotImplementedError("write your pallas kernel here")


# speedup vs baseline: 4.6276x; 4.6276x over previous
import functools

import jax
import jax.numpy as jnp
from jax import lax
from jax.experimental import pallas as pl
from jax.experimental.pallas import tpu as pltpu
from jax.experimental.pallas import tpu_sc as plsc

N = 10000
F = 128
E = 320000
NCLS = 919

NC = 2
NS = 16
NW = NC * NS
CH = 80
CPW = E // (NW * CH)
BR = 80
NB = N // BR
BPT = -(-NB // NS)

BN = 1000


def _sc_agg_body(support_hbm, src_hbm, dst_hbm, zeros_hbm, out_hbm,
                 idx_s, idx_d, rows_v, stage_v, acc, sem):
    c = lax.axis_index("c")
    s = lax.axis_index("s")
    wid = c * NS + s

    pltpu.sync_copy(zeros_hbm, stage_v)

    def zero_blk(i, _):
        j = i * NS + s

        @pl.when(j < NB)
        def _():
            pltpu.sync_copy(stage_v, acc.at[pl.ds(j * BR, BR), :])
        return 0

    lax.fori_loop(0, BPT, zero_blk, 0)
    plsc.subcore_barrier()

    def chunk(i, _):
        base = wid * (CPW * CH) + i * CH
        pltpu.sync_copy(src_hbm.at[pl.ds(base, CH)], idx_s)
        pltpu.sync_copy(dst_hbm.at[pl.ds(base, CH)], idx_d)
        pltpu.async_copy(support_hbm.at[idx_s], rows_v, sem).wait()
        pltpu.sync_copy(rows_v, acc.at[idx_d], add=True)
        return 0

    lax.fori_loop(0, CPW, chunk, 0)
    plsc.subcore_barrier()

    def drain_blk(i, _):
        j = i * NS + s

        @pl.when(j < NB)
        def _():
            pltpu.sync_copy(acc.at[pl.ds(j * BR, BR), :], stage_v)
            pltpu.sync_copy(stage_v, out_hbm.at[c, pl.ds(j * BR, BR), :])
        return 0

    lax.fori_loop(0, BPT, drain_blk, 0)


_sc_agg = functools.partial(
    pl.kernel,
    out_type=jax.ShapeDtypeStruct((NC, N, F), jnp.float32),
    mesh=plsc.VectorSubcoreMesh(
        core_axis_name="c", subcore_axis_name="s", num_cores=NC,
        num_subcores=NS),
    scratch_types=[
        pltpu.VMEM((CH,), jnp.int32),
        pltpu.VMEM((CH,), jnp.int32),
        pltpu.VMEM((CH, F), jnp.float32),
        pltpu.VMEM((BR, F), jnp.float32),
        pltpu.VMEM_SHARED((N, F), jnp.float32),
        pltpu.SemaphoreType.DMA,
    ],
)(_sc_agg_body)


def _mm_body(x_ref, w_ref, o_ref):
    o_ref[...] = jnp.dot(x_ref[...], w_ref[...],
                         preferred_element_type=jnp.float32)


def _support1(x, W):
    return pl.pallas_call(
        _mm_body,
        grid=(N // BN,),
        in_specs=[
            pl.BlockSpec((BN, F), lambda i: (i, 0)),
            pl.BlockSpec((F, F), lambda i: (0, 0)),
        ],
        out_specs=pl.BlockSpec((BN, F), lambda i: (i, 0)),
        out_shape=jax.ShapeDtypeStruct((N, F), jnp.float32),
    )(x, W)


def _gate_body(p_ref, q_ref, deg_ref, b1_ref, wg_ref, bg_ref, x_ref, w2_ref,
               g_ref, x1_ref, s2_ref):
    agg = (p_ref[0] + q_ref[0]) / deg_ref[...] + b1_ref[...]
    z = jnp.tanh(agg)
    g = jax.nn.sigmoid(jnp.dot(z, wg_ref[...],
                               preferred_element_type=jnp.float32)
                       + bg_ref[...])
    x1 = (1.0 - g) * x_ref[...] + g * z
    g_ref[...] = g
    x1_ref[...] = x1
    s2_ref[...] = jnp.dot(x1, w2_ref[...], preferred_element_type=jnp.float32)


def _gate_stage(parts, deg, b1, wg, bg, x, W2):
    return pl.pallas_call(
        _gate_body,
        grid=(N // BN,),
        in_specs=[
            pl.BlockSpec((1, BN, F), lambda i: (0, i, 0)),
            pl.BlockSpec((1, BN, F), lambda i: (1, i, 0)),
            pl.BlockSpec((BN, 1), lambda i: (i, 0)),
            pl.BlockSpec((1, F), lambda i: (0, 0)),
            pl.BlockSpec((F, 1), lambda i: (0, 0)),
            pl.BlockSpec((1, 1), lambda i: (0, 0)),
            pl.BlockSpec((BN, F), lambda i: (i, 0)),
            pl.BlockSpec((F, F), lambda i: (0, 0)),
        ],
        out_specs=[
            pl.BlockSpec((BN, 1), lambda i: (i, 0)),
            pl.BlockSpec((BN, F), lambda i: (i, 0)),
            pl.BlockSpec((BN, F), lambda i: (i, 0)),
        ],
        out_shape=[
            jax.ShapeDtypeStruct((N, 1), jnp.float32),
            jax.ShapeDtypeStruct((N, F), jnp.float32),
            jax.ShapeDtypeStruct((N, F), jnp.float32),
        ],
    )(parts, parts, deg, b1, wg, bg, x, W2)


def _final_body(p_ref, q_ref, deg_ref, b2_ref, wg_ref, bg_ref, x_ref,
                mu_ref, isg_ref, beta_ref, wo_ref, bo_ref,
                g_ref, out_ref):
    agg = (p_ref[0] + q_ref[0]) / deg_ref[...] + b2_ref[...]
    z = jnp.tanh(agg)
    g = jax.nn.sigmoid(jnp.dot(z, wg_ref[...],
                               preferred_element_type=jnp.float32)
                       + bg_ref[...])
    x2 = (1.0 - g) * x_ref[...] + g * z
    x2 = jnp.maximum(x2, 0.0)
    xb = (x2 - mu_ref[...]) * isg_ref[...] + beta_ref[...]
    g_ref[...] = g
    out_ref[...] = jnp.dot(xb, wo_ref[...],
                           preferred_element_type=jnp.float32) + bo_ref[...]


def _final_stage(parts, deg, b2, wg, bg, x1, mu, isg, beta, Wo, bo):
    return pl.pallas_call(
        _final_body,
        grid=(N // BN,),
        in_specs=[
            pl.BlockSpec((1, BN, F), lambda i: (0, i, 0)),
            pl.BlockSpec((1, BN, F), lambda i: (1, i, 0)),
            pl.BlockSpec((BN, 1), lambda i: (i, 0)),
            pl.BlockSpec((1, F), lambda i: (0, 0)),
            pl.BlockSpec((F, 1), lambda i: (0, 0)),
            pl.BlockSpec((1, 1), lambda i: (0, 0)),
            pl.BlockSpec((BN, F), lambda i: (i, 0)),
            pl.BlockSpec((1, F), lambda i: (0, 0)),
            pl.BlockSpec((1, F), lambda i: (0, 0)),
            pl.BlockSpec((1, F), lambda i: (0, 0)),
            pl.BlockSpec((F, NCLS), lambda i: (0, 0)),
            pl.BlockSpec((1, NCLS), lambda i: (0, 0)),
        ],
        out_specs=[
            pl.BlockSpec((BN, 1), lambda i: (i, 0)),
            pl.BlockSpec((BN, NCLS), lambda i: (i, 0)),
        ],
        out_shape=[
            jax.ShapeDtypeStruct((N, 1), jnp.float32),
            jax.ShapeDtypeStruct((N, NCLS), jnp.float32),
        ],
    )(parts, parts, deg, b2, wg, bg, x1, mu, isg, beta, Wo, bo)


def kernel(x_in, edge_index, deg, W_gc1, b_gc1, w_g1, b_g1, W_gc2, b_gc2,
           w_g2, b_g2, bn_gamma, bn_beta, bn_mean, bn_var, W_out, b_out):
    src = edge_index[0]
    dst = edge_index[1]
    zeros = jnp.zeros((BR, F), jnp.float32)

    support1 = _support1(x_in, W_gc1)
    parts1 = _sc_agg(support1, src, dst, zeros)
    g, x1, support2 = _gate_stage(
        parts1, deg, b_gc1.reshape(1, F), w_g1, b_g1.reshape(1, 1),
        x_in, W_gc2)

    parts2 = _sc_agg(support2, src, dst, zeros)
    inv_sigma = (bn_gamma / jnp.sqrt(bn_var + 1e-5)).reshape(1, F)
    g2, out = _final_stage(
        parts2, deg, b_gc2.reshape(1, F), w_g2, b_g2.reshape(1, 1),
        x1, bn_mean.reshape(1, F), inv_sigma, bn_beta.reshape(1, F),
        W_out, b_out.reshape(1, NCLS))

    return (x_in, out, g, g2)
